# baseline (speedup 1.0000x reference)
import functools

import jax
import jax.numpy as jnp
from jax import lax
from jax.experimental import pallas as pl
from jax.experimental.pallas import tpu as pltpu
from jax.experimental.pallas import tpu_sc as plsc

_CH = 128
_BI = 40
_R = 400


@functools.lru_cache(maxsize=None)
def _make_agg1(n_pad, d, n_chunks):
    cpt = n_chunks // 32
    rpt = n_pad // 16
    mesh = plsc.VectorSubcoreMesh(core_axis_name="c", subcore_axis_name="s")

    def body(t_lo, src2d, dst2d, out_lo, out_hi,
             acc, sidx, didx, rows_a, rows_b, gs_a, gs_b, ss_a, ss_b):
        cid = lax.axis_index("c")
        sid = lax.axis_index("s")

        def run(table, out):
            pltpu.sync_copy(table.at[pl.ds(sid * rpt, rpt)],
                            acc.at[pl.ds(sid * rpt, rpt)])
            plsc.subcore_barrier()
            cbase = (cid * 16 + sid) * cpt

            def fire_g(j, rows, sem):
                return pltpu.async_copy(table.at[sidx.at[j]], rows, sem)

            def fire_s(j, rows, sem):
                return pltpu.async_copy(rows, acc.at[didx.at[j]], sem, add=True)

            def blk_body(bk, carry):
                pltpu.sync_copy(src2d.at[pl.ds(cbase + bk * _BI, _BI)], sidx)
                pltpu.sync_copy(dst2d.at[pl.ds(cbase + bk * _BI, _BI)], didx)
                fire_g(0, rows_a, gs_a)
                fire_g(1, rows_b, gs_b)

                def pair(k, c2):
                    a = 2 * k
                    pltpu.make_async_copy(table.at[sidx.at[a]], rows_a, gs_a).wait()
                    fire_s(a, rows_a, ss_a)
                    pltpu.make_async_copy(rows_a, acc.at[didx.at[a]], ss_a).wait()
                    fire_g(a + 2, rows_a, gs_a)
                    pltpu.make_async_copy(table.at[sidx.at[a + 1]], rows_b, gs_b).wait()
                    fire_s(a + 1, rows_b, ss_b)
                    pltpu.make_async_copy(rows_b, acc.at[didx.at[a + 1]], ss_b).wait()
                    fire_g(a + 3, rows_b, gs_b)
                    return c2

                lax.fori_loop(0, _BI // 2 - 1, pair, 0)
                a = _BI - 2
                pltpu.make_async_copy(table.at[sidx.at[a]], rows_a, gs_a).wait()
                fire_s(a, rows_a, ss_a)
                pltpu.make_async_copy(table.at[sidx.at[a + 1]], rows_b, gs_b).wait()
                fire_s(a + 1, rows_b, ss_b)
                pltpu.make_async_copy(rows_a, acc.at[didx.at[a]], ss_a).wait()
                pltpu.make_async_copy(rows_b, acc.at[didx.at[a + 1]], ss_b).wait()
                return carry

            lax.fori_loop(0, cpt // _BI, blk_body, 0)
            plsc.subcore_barrier()
            pltpu.sync_copy(acc.at[pl.ds(sid * rpt, rpt)],
                            out.at[pl.ds(sid * rpt, rpt)])

        pl.when(cid == 0)(lambda: run(t_lo, out_lo))
        pl.when(cid == 1)(lambda: run(t_lo, out_hi))

    return pl.kernel(
        body,
        out_type=(jax.ShapeDtypeStruct((n_pad, d), jnp.float32),
                  jax.ShapeDtypeStruct((n_pad, d), jnp.float32)),
        mesh=mesh,
        scratch_types=[
            pltpu.VMEM_SHARED((n_pad, d), jnp.float32),
            pltpu.VMEM((_BI, _CH), jnp.int32),
            pltpu.VMEM((_BI, _CH), jnp.int32),
            pltpu.VMEM((_CH, d), jnp.float32),
            pltpu.VMEM((_CH, d), jnp.float32),
            pltpu.SemaphoreType.DMA,
            pltpu.SemaphoreType.DMA,
            pltpu.SemaphoreType.DMA,
            pltpu.SemaphoreType.DMA,
        ],
    )


@functools.lru_cache(maxsize=None)
def _make_part(n_chunks, q, cap):
    cpt = n_chunks // 32
    mesh = plsc.VectorSubcoreMesh(core_axis_name="c", subcore_axis_name="s")
    capp = cap + 16

    def body(src2d, dst2d, psrc, pdst, counts,
             sidx, didx, ls, ld, cvec):
        cid = lax.axis_index("c")
        sid = lax.axis_index("s")
        t = cid * 16 + sid
        base = t * cpt
        zero16 = jnp.zeros((16,), jnp.int32)
        junk16 = jnp.full((16,), q, dtype=jnp.int32)

        def pf(i, c):
            for sq in range(4):
                ls[pl.ds(sq * capp + i * 16, 16)] = zero16
                ld[pl.ds(sq * capp + i * 16, 16)] = junk16
            return c

        lax.fori_loop(0, capp // 16, pf, 0)

        offs = (jnp.int32(0), jnp.int32(0), jnp.int32(0), jnp.int32(0))
        for bk in range(cpt // _BI):
            pltpu.sync_copy(src2d.at[pl.ds(base + bk * _BI, _BI)], sidx)
            pltpu.sync_copy(dst2d.at[pl.ds(base + bk * _BI, _BI)], didx)

            def vec(v, o):
                r = v // 8
                l = (v % 8) * 16
                ss = sidx[r, pl.ds(l, 16)]
                dd = didx[r, pl.ds(l, 16)]
                no = []
                for sq in range(4):
                    lo = sq * q
                    m = jnp.logical_and(dd >= lo, dd < lo + q) if sq < 3 else (dd >= lo)
                    p = plsc.cumsum(m.astype(jnp.int32)) - 1 + o[sq] + sq * capp
                    plsc.store_scatter(ls, [p], ss, mask=m)
                    plsc.store_scatter(ld, [p], dd - lo, mask=m)
                    cnt = jnp.max(plsc.all_reduce_population_count(m))
                    no.append(o[sq] + cnt)
                return tuple(no)

            offs = lax.fori_loop(0, _BI * 8, vec, offs)

        iota = lax.iota(jnp.int32, 16)

        def cz(i, c):
            cvec[pl.ds(i * 16, 16)] = zero16
            return c

        lax.fori_loop(0, 8, cz, 0)
        cw = jnp.where(iota == 3, jnp.full((16,), offs[3], dtype=jnp.int32), 0)
        for sq in range(3):
            cw = jnp.where(iota == sq,
                           jnp.full((16,), offs[sq], dtype=jnp.int32), cw)
        cvec[pl.ds(0, 16)] = cw

        for sq in range(4):
            pltpu.sync_copy(ls.at[pl.ds(sq * capp, cap)],
                            psrc.at[pl.ds((sq * 32 + t) * cap, cap)])
            pltpu.sync_copy(ld.at[pl.ds(sq * capp, cap)],
                            pdst.at[pl.ds((sq * 32 + t) * cap, cap)])
        pltpu.sync_copy(cvec, counts.at[pl.ds(t * 128, 128)])

    return pl.kernel(
        body,
        out_type=(jax.ShapeDtypeStruct((128 * cap,), jnp.int32),
                  jax.ShapeDtypeStruct((128 * cap,), jnp.int32),
                  jax.ShapeDtypeStruct((32 * 128,), jnp.int32)),
        mesh=mesh,
        compiler_params=pltpu.CompilerParams(needs_layout_passes=False),
        scratch_types=[
            pltpu.VMEM((_BI, _CH), jnp.int32),
            pltpu.VMEM((_BI, _CH), jnp.int32),
            pltpu.VMEM((4 * capp,), jnp.int32),
            pltpu.VMEM((4 * capp,), jnp.int32),
            pltpu.VMEM((128,), jnp.int32),
        ],
    )


@functools.lru_cache(maxsize=None)
def _make_agg2(n_pad2, sa, q, cap, qbase):
    rpt = sa // 16
    spr = cap // _CH
    bi2 = 8
    mesh = plsc.VectorSubcoreMesh(core_axis_name="c", subcore_axis_name="s")

    def body(h1, psrc2d, pdst2d, counts, g_a, g_b,
             acc, sidx, didx, rows_a, rows_b, csm, gs_a, gs_b, ss_a, ss_b):
        cid = lax.axis_index("c")
        sid = lax.axis_index("s")

        def run(out):
            qq = qbase + cid
            pltpu.sync_copy(h1.at[pl.ds(qq * q + sid * rpt, rpt)],
                            acc.at[pl.ds(sid * rpt, rpt)])
            plsc.subcore_barrier()

            def fg(j, rows, sem):
                pltpu.async_copy(h1.at[sidx.at[j]], rows, sem)

            def wg(j, rows, sem):
                pltpu.make_async_copy(h1.at[sidx.at[j]], rows, sem).wait()

            def fs(j, rows, sem):
                pltpu.async_copy(rows, acc.at[didx.at[j]], sem, add=True)

            def ws(j, rows, sem):
                pltpu.make_async_copy(rows, acc.at[didx.at[j]], sem).wait()

            for sg in range(2):
                srct = 2 * sid + sg
                segrow = (qq * 32 + srct) * spr
                pltpu.sync_copy(counts.at[pl.ds(srct * 128, 32)], csm)
                cv = csm[pl.ds(qq, 16)]
                cnt = cv[0]
                n_rows = jnp.maximum((cnt + 127) // 128, 1)
                nb = (n_rows + bi2 - 1) // bi2

                def blk(bk, c0):
                    pltpu.sync_copy(psrc2d.at[pl.ds(segrow + bk * bi2, bi2)],
                                    sidx)
                    pltpu.sync_copy(pdst2d.at[pl.ds(segrow + bk * bi2, bi2)],
                                    didx)
                    fg(0, rows_a, gs_a)
                    fg(1, rows_b, gs_b)

                    def pair(k, cc):
                        a = 2 * k
                        wg(a, rows_a, gs_a)
                        fs(a, rows_a, ss_a)
                        ws(a, rows_a, ss_a)
                        fg(a + 2, rows_a, gs_a)
                        wg(a + 1, rows_b, gs_b)
                        fs(a + 1, rows_b, ss_b)
                        ws(a + 1, rows_b, ss_b)
                        fg(a + 3, rows_b, gs_b)
                        return cc

                    lax.fori_loop(0, bi2 // 2 - 1, pair, 0)
                    rl = bi2 - 2
                    wg(rl, rows_a, gs_a)
                    fs(rl, rows_a, ss_a)
                    wg(rl + 1, rows_b, gs_b)
                    fs(rl + 1, rows_b, ss_b)
                    ws(rl, rows_a, ss_a)
                    ws(rl + 1, rows_b, ss_b)
                    return c0

                lax.fori_loop(0, nb, blk, 0)

            plsc.subcore_barrier()
            pltpu.sync_copy(acc.at[pl.ds(sid * rpt, rpt)],
                            out.at[pl.ds(sid * rpt, rpt)])

        pl.when(cid == 0)(lambda: run(g_a))
        pl.when(cid == 1)(lambda: run(g_b))

    return pl.kernel(
        body,
        out_type=(jax.ShapeDtypeStruct((sa, 2, 128), jnp.float32),
                  jax.ShapeDtypeStruct((sa, 2, 128), jnp.float32)),
        mesh=mesh,
        scratch_types=[
            pltpu.VMEM_SHARED((sa, 2, 128), jnp.float32),
            pltpu.VMEM((8, _CH), jnp.int32),
            pltpu.VMEM((8, _CH), jnp.int32),
            pltpu.VMEM((_CH, 2, 128), jnp.float32),
            pltpu.VMEM((_CH, 2, 128), jnp.float32),
            pltpu.VMEM((32,), jnp.int32),
            pltpu.SemaphoreType.DMA,
            pltpu.SemaphoreType.DMA,
            pltpu.SemaphoreType.DMA,
            pltpu.SemaphoreType.DMA,
        ],
    )


def _mlp1(n, n_pad2, a0, a1, x, w_a, b_a, w_b, b_b, eps):
    din = w_a.shape[0]
    dout = w_b.shape[1]
    ce = eps - 1.0

    def body(a0r, a1r, xr, wa, ba, wb, bb, o):
        h0 = a0r[...] + a1r[...] + ce * xr[...]
        z = jnp.dot(h0, wa[...], preferred_element_type=jnp.float32) + ba[...]
        z = jnp.maximum(z, 0.0)
        z = jnp.dot(z, wb[...], preferred_element_type=jnp.float32) + bb[...]
        o[...] = jnp.maximum(z, 0.0)

    row_spec = lambda cols: pl.BlockSpec((_R, cols), lambda i: (i, 0))
    full = lambda arr: pl.BlockSpec(arr.shape, lambda i: (0,) * arr.ndim)
    return pl.pallas_call(
        body,
        grid=(n // _R,),
        in_specs=[row_spec(din), row_spec(din), row_spec(din),
                  full(w_a), full(b_a), full(w_b), full(b_b)],
        out_specs=pl.BlockSpec((_R, dout), lambda i: (i, 0)),
        out_shape=jax.ShapeDtypeStruct((n_pad2, dout), jnp.float32),
    )(a0, a1, x, w_a, b_a, w_b, b_b)


def _mlp2(n, q, g, h1, w_a, b_a, w_b, b_b, eps):
    dout = w_b.shape[1]
    hd = g[0].shape[1]
    tq = q // _R

    def body(g0, g1, g2, g3, hr, wa, ba, wb, bb, o):
        i = pl.program_id(0)
        qq = i // tq
        h0 = jnp.where(qq == 0, g0[...],
                       jnp.where(qq == 1, g1[...],
                                 jnp.where(qq == 2, g2[...], g3[...])))
        h0 = h0 + eps * hr[...]
        z = jnp.dot(h0, wa[...], preferred_element_type=jnp.float32) + ba[...]
        z = jnp.maximum(z, 0.0)
        o[...] = jnp.dot(z, wb[...], preferred_element_type=jnp.float32) + bb[...]

    full = lambda arr: pl.BlockSpec(arr.shape, lambda i: (0,) * arr.ndim)

    def qspec(sq):
        def imap(i):
            return (jnp.clip(i - sq * tq, 0, tq - 1), 0)
        return pl.BlockSpec((_R, hd), imap)

    return pl.pallas_call(
        body,
        grid=(n // _R,),
        in_specs=[qspec(0), qspec(1), qspec(2), qspec(3),
                  pl.BlockSpec((_R, hd), lambda i: (i, 0)),
                  full(w_a), full(b_a), full(w_b), full(b_b)],
        out_specs=pl.BlockSpec((_R, dout), lambda i: (i, 0)),
        out_shape=jax.ShapeDtypeStruct((n, dout), jnp.float32),
    )(g[0], g[1], g[2], g[3], h1, w_a, b_a, w_b, b_b)


def kernel(x, edge_index, W1, b1, W2, b2, W3, b3, W4, b4):
    n, f_in = x.shape
    e = edge_index.shape[1]
    src = edge_index[0].astype(jnp.int32)
    dst = edge_index[1].astype(jnp.int32)

    e_pad = -(-e // (32 * 8 * _CH)) * (32 * 8 * _CH)
    n_pad = -(-n // 128) * 128
    q = -(-(n // 4) // _R) * _R
    sa = -(-(q + 8) // 128) * 128
    n_pad2 = -(-(3 * q + sa) // 128) * 128
    pad = e_pad - e
    src2d = jnp.concatenate([src, jnp.zeros((pad,), jnp.int32)]).reshape(-1, _CH)
    dst2d = jnp.concatenate([dst, jnp.full((pad,), n, jnp.int32)]).reshape(-1, _CH)
    n_chunks = e_pad // _CH
    cap = e_pad // 32

    def row_pad(arr, rows):
        return jnp.concatenate(
            [arr, jnp.zeros((rows - arr.shape[0], arr.shape[1]), arr.dtype)])

    x_p = row_pad(x, n_pad)
    b1r, b2r, b3r, b4r = (b.reshape(1, -1) for b in (b1, b2, b3, b4))

    psrc, pdst, counts = _make_part(n_chunks, q, cap)(src2d, dst2d)
    a0, a1 = _make_agg1(n_pad, f_in, n_chunks)(x_p, src2d, dst2d)
    h1 = _mlp1(n, n_pad2, a0, a1, x_p, W1, b1r, W2, b2r, 1e-09)
    psrc2d = psrc.reshape(-1, _CH)
    pdst2d = pdst.reshape(-1, _CH)
    h1w = h1.reshape(n_pad2, 2, 128)
    g0, g1 = _make_agg2(n_pad2, sa, q, cap, 0)(h1w, psrc2d, pdst2d, counts)
    g2, g3 = _make_agg2(n_pad2, sa, q, cap, 2)(h1w, psrc2d, pdst2d, counts)
    g = tuple(t.reshape(sa, 256) for t in (g0, g1, g2, g3))
    out = _mlp2(n, q, g, h1, W3, b3r, W4, b4r, 1e-13)
    return out


# speedup vs baseline: 1.0434x; 1.0434x over previous
import functools

import jax
import jax.numpy as jnp
from jax import lax
from jax.experimental import pallas as pl
from jax.experimental.pallas import tpu as pltpu
from jax.experimental.pallas import tpu_sc as plsc

_CH = 128
_BI = 40
_R = 400


@functools.lru_cache(maxsize=None)
def _make_agg1(n_pad, d, n_chunks):
    cpt = n_chunks // 32
    rpt = n_pad // 16
    mesh = plsc.VectorSubcoreMesh(core_axis_name="c", subcore_axis_name="s")

    def body(t_lo, src2d, dst2d, out_lo, out_hi,
             acc, sidx, didx, rows_a, rows_b, gs_a, gs_b, ss_a, ss_b):
        cid = lax.axis_index("c")
        sid = lax.axis_index("s")

        def run(table, out):
            pltpu.sync_copy(table.at[pl.ds(sid * rpt, rpt)],
                            acc.at[pl.ds(sid * rpt, rpt)])
            plsc.subcore_barrier()
            cbase = (cid * 16 + sid) * cpt

            def fire_g(j, rows, sem):
                return pltpu.async_copy(table.at[sidx.at[j]], rows, sem)

            def fire_s(j, rows, sem):
                return pltpu.async_copy(rows, acc.at[didx.at[j]], sem, add=True)

            def blk_body(bk, carry):
                pltpu.sync_copy(src2d.at[pl.ds(cbase + bk * _BI, _BI)], sidx)
                pltpu.sync_copy(dst2d.at[pl.ds(cbase + bk * _BI, _BI)], didx)
                fire_g(0, rows_a, gs_a)
                fire_g(1, rows_b, gs_b)

                def pair(k, c2):
                    a = 2 * k
                    pltpu.make_async_copy(table.at[sidx.at[a]], rows_a, gs_a).wait()
                    fire_s(a, rows_a, ss_a)
                    pltpu.make_async_copy(rows_a, acc.at[didx.at[a]], ss_a).wait()
                    fire_g(a + 2, rows_a, gs_a)
                    pltpu.make_async_copy(table.at[sidx.at[a + 1]], rows_b, gs_b).wait()
                    fire_s(a + 1, rows_b, ss_b)
                    pltpu.make_async_copy(rows_b, acc.at[didx.at[a + 1]], ss_b).wait()
                    fire_g(a + 3, rows_b, gs_b)
                    return c2

                lax.fori_loop(0, _BI // 2 - 1, pair, 0)
                a = _BI - 2
                pltpu.make_async_copy(table.at[sidx.at[a]], rows_a, gs_a).wait()
                fire_s(a, rows_a, ss_a)
                pltpu.make_async_copy(table.at[sidx.at[a + 1]], rows_b, gs_b).wait()
                fire_s(a + 1, rows_b, ss_b)
                pltpu.make_async_copy(rows_a, acc.at[didx.at[a]], ss_a).wait()
                pltpu.make_async_copy(rows_b, acc.at[didx.at[a + 1]], ss_b).wait()
                return carry

            lax.fori_loop(0, cpt // _BI, blk_body, 0)
            plsc.subcore_barrier()
            pltpu.sync_copy(acc.at[pl.ds(sid * rpt, rpt)],
                            out.at[pl.ds(sid * rpt, rpt)])

        pl.when(cid == 0)(lambda: run(t_lo, out_lo))
        pl.when(cid == 1)(lambda: run(t_lo, out_hi))

    return pl.kernel(
        body,
        out_type=(jax.ShapeDtypeStruct((n_pad, d), jnp.float32),
                  jax.ShapeDtypeStruct((n_pad, d), jnp.float32)),
        mesh=mesh,
        scratch_types=[
            pltpu.VMEM_SHARED((n_pad, d), jnp.float32),
            pltpu.VMEM((_BI, _CH), jnp.int32),
            pltpu.VMEM((_BI, _CH), jnp.int32),
            pltpu.VMEM((_CH, d), jnp.float32),
            pltpu.VMEM((_CH, d), jnp.float32),
            pltpu.SemaphoreType.DMA,
            pltpu.SemaphoreType.DMA,
            pltpu.SemaphoreType.DMA,
            pltpu.SemaphoreType.DMA,
        ],
    )


@functools.lru_cache(maxsize=None)
def _make_part(n_chunks, q, cap):
    cpt = n_chunks // 32
    mesh = plsc.VectorSubcoreMesh(core_axis_name="c", subcore_axis_name="s")
    capp = cap + 16

    def body(src2d, dst2d, psrc, pdst, counts,
             sidx, didx, ls, ld, cvec):
        cid = lax.axis_index("c")
        sid = lax.axis_index("s")
        t = cid * 16 + sid
        base = t * cpt
        zero16 = jnp.zeros((16,), jnp.int32)
        junk16 = q + lax.iota(jnp.int32, 16)

        def pf(i, c):
            for sq in range(4):
                ls[pl.ds(sq * capp + i * 16, 16)] = zero16
                ld[pl.ds(sq * capp + i * 16, 16)] = junk16
            return c

        lax.fori_loop(0, capp // 16, pf, 0)

        offs = (jnp.int32(0), jnp.int32(0), jnp.int32(0), jnp.int32(0))
        for bk in range(cpt // _BI):
            pltpu.sync_copy(src2d.at[pl.ds(base + bk * _BI, _BI)], sidx)
            pltpu.sync_copy(dst2d.at[pl.ds(base + bk * _BI, _BI)], didx)

            def vec(v, o):
                r = v // 8
                l = (v % 8) * 16
                ss = sidx[r, pl.ds(l, 16)]
                dd = didx[r, pl.ds(l, 16)]
                no = []
                for sq in range(4):
                    lo = sq * q
                    m = jnp.logical_and(dd >= lo, dd < lo + q) if sq < 3 else (dd >= lo)
                    p = plsc.cumsum(m.astype(jnp.int32)) - 1 + o[sq] + sq * capp
                    plsc.store_scatter(ls, [p], ss, mask=m)
                    plsc.store_scatter(ld, [p], dd - lo, mask=m)
                    cnt = jnp.max(plsc.all_reduce_population_count(m))
                    no.append(o[sq] + cnt)
                return tuple(no)

            offs = lax.fori_loop(0, _BI * 8, vec, offs)

        iota = lax.iota(jnp.int32, 16)

        def cz(i, c):
            cvec[pl.ds(i * 16, 16)] = zero16
            return c

        lax.fori_loop(0, 8, cz, 0)
        cw = jnp.where(iota == 3, jnp.full((16,), offs[3], dtype=jnp.int32), 0)
        for sq in range(3):
            cw = jnp.where(iota == sq,
                           jnp.full((16,), offs[sq], dtype=jnp.int32), cw)
        cvec[pl.ds(0, 16)] = cw

        for sq in range(4):
            pltpu.sync_copy(ls.at[pl.ds(sq * capp, cap)],
                            psrc.at[pl.ds((sq * 32 + t) * cap, cap)])
            pltpu.sync_copy(ld.at[pl.ds(sq * capp, cap)],
                            pdst.at[pl.ds((sq * 32 + t) * cap, cap)])
        pltpu.sync_copy(cvec, counts.at[pl.ds(t * 128, 128)])

    return pl.kernel(
        body,
        out_type=(jax.ShapeDtypeStruct((128 * cap,), jnp.int32),
                  jax.ShapeDtypeStruct((128 * cap,), jnp.int32),
                  jax.ShapeDtypeStruct((32 * 128,), jnp.int32)),
        mesh=mesh,
        compiler_params=pltpu.CompilerParams(needs_layout_passes=False),
        scratch_types=[
            pltpu.VMEM((_BI, _CH), jnp.int32),
            pltpu.VMEM((_BI, _CH), jnp.int32),
            pltpu.VMEM((4 * capp,), jnp.int32),
            pltpu.VMEM((4 * capp,), jnp.int32),
            pltpu.VMEM((128,), jnp.int32),
        ],
    )


@functools.lru_cache(maxsize=None)
def _make_agg2(n_pad2, sa, q, cap, qbase):
    rpt = sa // 16
    spr = cap // _CH
    bi2 = 8
    mesh = plsc.VectorSubcoreMesh(core_axis_name="c", subcore_axis_name="s")

    def body(h1, psrc2d, pdst2d, counts, g_a, g_b,
             acc, sidx, didx, rows_a, rows_b, csm, gs_a, gs_b, ss_a, ss_b):
        cid = lax.axis_index("c")
        sid = lax.axis_index("s")

        def run(out):
            qq = qbase + cid
            pltpu.sync_copy(h1.at[pl.ds(qq * q + sid * rpt, rpt)],
                            acc.at[pl.ds(sid * rpt, rpt)])
            plsc.subcore_barrier()

            def fg(j, rows, sem):
                pltpu.async_copy(h1.at[sidx.at[j]], rows, sem)

            def wg(j, rows, sem):
                pltpu.make_async_copy(h1.at[sidx.at[j]], rows, sem).wait()

            def fs(j, rows, sem):
                pltpu.async_copy(rows, acc.at[didx.at[j]], sem, add=True)

            def ws(j, rows, sem):
                pltpu.make_async_copy(rows, acc.at[didx.at[j]], sem).wait()

            for sg in range(2):
                srct = 2 * sid + sg
                segrow = (qq * 32 + srct) * spr
                pltpu.sync_copy(counts.at[pl.ds(srct * 128, 32)], csm)
                cv = csm[pl.ds(qq, 16)]
                cnt = cv[0]
                n_rows = jnp.maximum((cnt + 127) // 128, 1)
                nb = (n_rows + bi2 - 1) // bi2

                def blk(bk, c0):
                    pltpu.sync_copy(psrc2d.at[pl.ds(segrow + bk * bi2, bi2)],
                                    sidx)
                    pltpu.sync_copy(pdst2d.at[pl.ds(segrow + bk * bi2, bi2)],
                                    didx)
                    fg(0, rows_a, gs_a)
                    fg(1, rows_b, gs_b)

                    def pair(k, cc):
                        a = 2 * k
                        wg(a, rows_a, gs_a)
                        fs(a, rows_a, ss_a)
                        ws(a, rows_a, ss_a)
                        fg(a + 2, rows_a, gs_a)
                        wg(a + 1, rows_b, gs_b)
                        fs(a + 1, rows_b, ss_b)
                        ws(a + 1, rows_b, ss_b)
                        fg(a + 3, rows_b, gs_b)
                        return cc

                    lax.fori_loop(0, bi2 // 2 - 1, pair, 0)
                    rl = bi2 - 2
                    wg(rl, rows_a, gs_a)
                    fs(rl, rows_a, ss_a)
                    wg(rl + 1, rows_b, gs_b)
                    fs(rl + 1, rows_b, ss_b)
                    ws(rl, rows_a, ss_a)
                    ws(rl + 1, rows_b, ss_b)
                    return c0

                lax.fori_loop(0, nb, blk, 0)

            plsc.subcore_barrier()
            pltpu.sync_copy(acc.at[pl.ds(sid * rpt, rpt)],
                            out.at[pl.ds(sid * rpt, rpt)])

        pl.when(cid == 0)(lambda: run(g_a))
        pl.when(cid == 1)(lambda: run(g_b))

    return pl.kernel(
        body,
        out_type=(jax.ShapeDtypeStruct((sa, 2, 128), jnp.float32),
                  jax.ShapeDtypeStruct((sa, 2, 128), jnp.float32)),
        mesh=mesh,
        scratch_types=[
            pltpu.VMEM_SHARED((sa, 2, 128), jnp.float32),
            pltpu.VMEM((8, _CH), jnp.int32),
            pltpu.VMEM((8, _CH), jnp.int32),
            pltpu.VMEM((_CH, 2, 128), jnp.float32),
            pltpu.VMEM((_CH, 2, 128), jnp.float32),
            pltpu.VMEM((32,), jnp.int32),
            pltpu.SemaphoreType.DMA,
            pltpu.SemaphoreType.DMA,
            pltpu.SemaphoreType.DMA,
            pltpu.SemaphoreType.DMA,
        ],
    )


def _mlp1(n, n_pad2, a0, a1, x, w_a, b_a, w_b, b_b, eps):
    din = w_a.shape[0]
    dout = w_b.shape[1]
    ce = eps - 1.0

    def body(a0r, a1r, xr, wa, ba, wb, bb, o):
        h0 = a0r[...] + a1r[...] + ce * xr[...]
        z = jnp.dot(h0, wa[...], preferred_element_type=jnp.float32) + ba[...]
        z = jnp.maximum(z, 0.0)
        z = jnp.dot(z, wb[...], preferred_element_type=jnp.float32) + bb[...]
        o[...] = jnp.maximum(z, 0.0)

    row_spec = lambda cols: pl.BlockSpec((_R, cols), lambda i: (i, 0))
    full = lambda arr: pl.BlockSpec(arr.shape, lambda i: (0,) * arr.ndim)
    return pl.pallas_call(
        body,
        grid=(n // _R,),
        in_specs=[row_spec(din), row_spec(din), row_spec(din),
                  full(w_a), full(b_a), full(w_b), full(b_b)],
        out_specs=pl.BlockSpec((_R, dout), lambda i: (i, 0)),
        out_shape=jax.ShapeDtypeStruct((n_pad2, dout), jnp.float32),
    )(a0, a1, x, w_a, b_a, w_b, b_b)


def _mlp2(n, q, g, h1, w_a, b_a, w_b, b_b, eps):
    dout = w_b.shape[1]
    hd = g[0].shape[1]
    tq = q // _R

    def body(g0, g1, g2, g3, hr, wa, ba, wb, bb, o):
        i = pl.program_id(0)
        qq = i // tq
        h0 = jnp.where(qq == 0, g0[...],
                       jnp.where(qq == 1, g1[...],
                                 jnp.where(qq == 2, g2[...], g3[...])))
        h0 = h0 + eps * hr[...]
        z = jnp.dot(h0, wa[...], preferred_element_type=jnp.float32) + ba[...]
        z = jnp.maximum(z, 0.0)
        o[...] = jnp.dot(z, wb[...], preferred_element_type=jnp.float32) + bb[...]

    full = lambda arr: pl.BlockSpec(arr.shape, lambda i: (0,) * arr.ndim)

    def qspec(sq):
        def imap(i):
            return (jnp.clip(i - sq * tq, 0, tq - 1), 0)
        return pl.BlockSpec((_R, hd), imap)

    return pl.pallas_call(
        body,
        grid=(n // _R,),
        in_specs=[qspec(0), qspec(1), qspec(2), qspec(3),
                  pl.BlockSpec((_R, hd), lambda i: (i, 0)),
                  full(w_a), full(b_a), full(w_b), full(b_b)],
        out_specs=pl.BlockSpec((_R, dout), lambda i: (i, 0)),
        out_shape=jax.ShapeDtypeStruct((n, dout), jnp.float32),
    )(g[0], g[1], g[2], g[3], h1, w_a, b_a, w_b, b_b)


def kernel(x, edge_index, W1, b1, W2, b2, W3, b3, W4, b4):
    n, f_in = x.shape
    e = edge_index.shape[1]
    src = edge_index[0].astype(jnp.int32)
    dst = edge_index[1].astype(jnp.int32)

    e_pad = -(-e // (32 * 8 * _CH)) * (32 * 8 * _CH)
    n_pad = -(-n // 128) * 128
    q = -(-(n // 4) // _R) * _R
    sa = -(-(q + 8) // 128) * 128
    n_pad2 = -(-(3 * q + sa) // 128) * 128
    pad = e_pad - e
    src2d = jnp.concatenate([src, jnp.zeros((pad,), jnp.int32)]).reshape(-1, _CH)
    pad_dst = n + jnp.arange(pad, dtype=jnp.int32) % jnp.maximum(n_pad - n, 1)
    dst2d = jnp.concatenate([dst, pad_dst]).reshape(-1, _CH)
    n_chunks = e_pad // _CH
    cap = e_pad // 32

    def row_pad(arr, rows):
        return jnp.concatenate(
            [arr, jnp.zeros((rows - arr.shape[0], arr.shape[1]), arr.dtype)])

    x_p = row_pad(x, n_pad)
    b1r, b2r, b3r, b4r = (b.reshape(1, -1) for b in (b1, b2, b3, b4))

    psrc, pdst, counts = _make_part(n_chunks, q, cap)(src2d, dst2d)
    a0, a1 = _make_agg1(n_pad, f_in, n_chunks)(x_p, src2d, dst2d)
    h1 = _mlp1(n, n_pad2, a0, a1, x_p, W1, b1r, W2, b2r, 1e-09)
    psrc2d = psrc.reshape(-1, _CH)
    pdst2d = pdst.reshape(-1, _CH)
    h1w = h1.reshape(n_pad2, 2, 128)
    g0, g1 = _make_agg2(n_pad2, sa, q, cap, 0)(h1w, psrc2d, pdst2d, counts)
    g2, g3 = _make_agg2(n_pad2, sa, q, cap, 2)(h1w, psrc2d, pdst2d, counts)
    g = tuple(t.reshape(sa, 256) for t in (g0, g1, g2, g3))
    out = _mlp2(n, q, g, h1, W3, b3r, W4, b4r, 1e-13)
    return out


# speedup vs baseline: 2.3159x; 2.2195x over previous
import functools

import jax
import jax.numpy as jnp
from jax import lax
from jax.experimental import pallas as pl
from jax.experimental.pallas import tpu as pltpu
from jax.experimental.pallas import tpu_sc as plsc

_CH = 128
_BI = 40


@functools.lru_cache(maxsize=None)
def _make_agg(n_pad, d, n_chunks, edge_split):
    cpt = n_chunks // (32 if edge_split else 16)
    rpt = n_pad // 16
    mesh = plsc.VectorSubcoreMesh(core_axis_name="c", subcore_axis_name="s")

    def body(t_lo, t_hi, src2d, dst2d, out_lo, out_hi,
             acc, sidx, didx, rows_a, rows_b, gs_a, gs_b, ss_a, ss_b):
        cid = lax.axis_index("c")
        sid = lax.axis_index("s")

        def run(table, out):
            pltpu.sync_copy(table.at[pl.ds(sid * rpt, rpt)],
                            acc.at[pl.ds(sid * rpt, rpt)])
            plsc.subcore_barrier()
            if edge_split:
                cbase = (cid * 16 + sid) * cpt
            else:
                cbase = sid * cpt

            def fire_g(j, rows, sem):
                return pltpu.async_copy(table.at[sidx.at[j]], rows, sem)

            def fire_s(j, rows, sem):
                return pltpu.async_copy(rows, acc.at[didx.at[j]], sem, add=True)

            def blk_body(bk, carry):
                pltpu.sync_copy(src2d.at[pl.ds(cbase + bk * _BI, _BI)], sidx)
                pltpu.sync_copy(dst2d.at[pl.ds(cbase + bk * _BI, _BI)], didx)
                fire_g(0, rows_a, gs_a)
                fire_g(1, rows_b, gs_b)

                def pair(k, c2):
                    a = 2 * k
                    pltpu.make_async_copy(table.at[sidx.at[a]], rows_a, gs_a).wait()
                    fire_s(a, rows_a, ss_a)
                    pltpu.make_async_copy(rows_a, acc.at[didx.at[a]], ss_a).wait()
                    fire_g(a + 2, rows_a, gs_a)
                    pltpu.make_async_copy(table.at[sidx.at[a + 1]], rows_b, gs_b).wait()
                    fire_s(a + 1, rows_b, ss_b)
                    pltpu.make_async_copy(rows_b, acc.at[didx.at[a + 1]], ss_b).wait()
                    fire_g(a + 3, rows_b, gs_b)
                    return c2

                lax.fori_loop(0, _BI // 2 - 1, pair, 0)
                a = _BI - 2
                pltpu.make_async_copy(table.at[sidx.at[a]], rows_a, gs_a).wait()
                fire_s(a, rows_a, ss_a)
                pltpu.make_async_copy(table.at[sidx.at[a + 1]], rows_b, gs_b).wait()
                fire_s(a + 1, rows_b, ss_b)
                pltpu.make_async_copy(rows_a, acc.at[didx.at[a]], ss_a).wait()
                pltpu.make_async_copy(rows_b, acc.at[didx.at[a + 1]], ss_b).wait()
                return carry

            lax.fori_loop(0, cpt // _BI, blk_body, 0)
            plsc.subcore_barrier()
            pltpu.sync_copy(acc.at[pl.ds(sid * rpt, rpt)],
                            out.at[pl.ds(sid * rpt, rpt)])

        if edge_split:
            pl.when(cid == 0)(lambda: run(t_lo, out_lo))
            pl.when(cid == 1)(lambda: run(t_lo, out_hi))
        else:
            pl.when(cid == 0)(lambda: run(t_lo, out_lo))
            pl.when(cid == 1)(lambda: run(t_hi, out_hi))

    return pl.kernel(
        body,
        out_type=(jax.ShapeDtypeStruct((n_pad, d), jnp.float32),
                  jax.ShapeDtypeStruct((n_pad, d), jnp.float32)),
        mesh=mesh,
        scratch_types=[
            pltpu.VMEM_SHARED((n_pad, d), jnp.float32),
            pltpu.VMEM((_BI, _CH), jnp.int32),
            pltpu.VMEM((_BI, _CH), jnp.int32),
            pltpu.VMEM((_CH, d), jnp.float32),
            pltpu.VMEM((_CH, d), jnp.float32),
            pltpu.SemaphoreType.DMA,
            pltpu.SemaphoreType.DMA,
            pltpu.SemaphoreType.DMA,
            pltpu.SemaphoreType.DMA,
        ],
    )


def _row_tile(n):
    for r in (512, 400, 256, 200, 128, 80, 40, 16, 8):
        if n % r == 0:
            return r
    return 1


def _mlp1(n, a0, a1, x, w_a, b_a, w_b, b_b, eps):
    n_pad = a0.shape[0]
    din = w_a.shape[0]
    dout = w_b.shape[1]
    r = _row_tile(n)
    ce = eps - 1.0

    def body(a0r, a1r, xr, wa, ba, wb, bb, o_lo, o_hi):
        h0 = a0r[...] + a1r[...] + ce * xr[...]
        z = jnp.dot(h0, wa[...], preferred_element_type=jnp.float32) + ba[...]
        z = jnp.maximum(z, 0.0)
        z = jnp.dot(z, wb[...], preferred_element_type=jnp.float32) + bb[...]
        z = jnp.maximum(z, 0.0)
        o_lo[...] = z[:, : dout // 2]
        o_hi[...] = z[:, dout // 2:]

    row_spec = lambda cols: pl.BlockSpec((r, cols), lambda i: (i, 0))
    full = lambda arr: pl.BlockSpec(arr.shape, lambda i: (0,) * arr.ndim)
    return pl.pallas_call(
        body,
        grid=(n // r,),
        in_specs=[row_spec(din), row_spec(din), row_spec(din),
                  full(w_a), full(b_a), full(w_b), full(b_b)],
        out_specs=[row_spec(dout // 2), row_spec(dout // 2)],
        out_shape=(jax.ShapeDtypeStruct((n_pad, dout // 2), jnp.float32),
                   jax.ShapeDtypeStruct((n_pad, dout // 2), jnp.float32)),
    )(a0, a1, x, w_a, b_a, w_b, b_b)


def _mlp2(n, g_lo, g_hi, h_lo, h_hi, w_a, b_a, w_b, b_b, eps):
    dout = w_b.shape[1]
    hd = g_lo.shape[1]
    r = _row_tile(n)

    def body(gl, gh, hl, hh, wa, ba, wb, bb, o):
        h0 = jnp.concatenate([gl[...] + eps * hl[...],
                              gh[...] + eps * hh[...]], axis=1)
        z = jnp.dot(h0, wa[...], preferred_element_type=jnp.float32) + ba[...]
        z = jnp.maximum(z, 0.0)
        o[...] = jnp.dot(z, wb[...], preferred_element_type=jnp.float32) + bb[...]

    row_spec = lambda cols: pl.BlockSpec((r, cols), lambda i: (i, 0))
    full = lambda arr: pl.BlockSpec(arr.shape, lambda i: (0,) * arr.ndim)
    return pl.pallas_call(
        body,
        grid=(n // r,),
        in_specs=[row_spec(hd), row_spec(hd), row_spec(hd), row_spec(hd),
                  full(w_a), full(b_a), full(w_b), full(b_b)],
        out_specs=pl.BlockSpec((r, dout), lambda i: (i, 0)),
        out_shape=jax.ShapeDtypeStruct((n, dout), jnp.float32),
    )(g_lo, g_hi, h_lo, h_hi, w_a, b_a, w_b, b_b)


def kernel(x, edge_index, W1, b1, W2, b2, W3, b3, W4, b4):
    n, f_in = x.shape
    e = edge_index.shape[1]
    src = edge_index[0].astype(jnp.int32)
    dst = edge_index[1].astype(jnp.int32)

    e_pad = -(-e // (32 * 8 * _CH)) * (32 * 8 * _CH)
    n_pad = -(-n // 128) * 128
    pad = e_pad - e
    src2d = jnp.concatenate([src, jnp.zeros((pad,), jnp.int32)]).reshape(-1, _CH)
    dst2d = jnp.concatenate([dst, jnp.full((pad,), n, jnp.int32)]).reshape(-1, _CH)
    n_chunks = e_pad // _CH

    def row_pad(arr):
        return jnp.concatenate([arr, jnp.zeros((n_pad - n, arr.shape[1]), arr.dtype)])

    x_p = row_pad(x)
    b1r, b2r, b3r, b4r = (b.reshape(1, -1) for b in (b1, b2, b3, b4))

    a0, a1 = _make_agg(n_pad, f_in, n_chunks, True)(x_p, x_p, src2d, dst2d)
    h_lo, h_hi = _mlp1(n, a0, a1, x_p, W1, b1r, W2, b2r, 1e-09)
    hh = W2.shape[1] // 2
    g_lo, g_hi = _make_agg(n_pad, hh, n_chunks, False)(h_lo, h_hi, src2d, dst2d)
    out = _mlp2(n, g_lo, g_hi, h_lo, h_hi, W3, b3r, W4, b4r, 1e-13)
    return out
